# baseline (speedup 1.0000x reference)
import functools

import jax
import jax.numpy as jnp
from jax import lax
from jax.experimental import pallas as pl
from jax.experimental.pallas import tpu as pltpu
from jax.experimental.pallas import tpu_sc as plsc

PATCH = 16
FEAT = 24
NP_ = FEAT * FEAT
CUT = NP_ // 2
IMG = FEAT * PATCH
RB = 128
NRB = IMG // RB
COLW = RB * IMG
NCHUNK = NP_ // 16
NHEAD = 12


def _sel_sc_body(xr_hbm, sel_hbm, xbuf, kbuf, sbuf, cbuf):
    wid = lax.axis_index("s") * 2 + lax.axis_index("c")

    @pl.when(wid < xr_hbm.shape[0])
    def _():
        b = wid
        pltpu.sync_copy(xr_hbm.at[b], xbuf)

        def score_chunk(i, carry):
            acc = xbuf[0, pl.ds(i * 16, 16)]
            for h in range(1, NHEAD):
                acc = acc + xbuf[h, pl.ds(i * 16, 16)]
            bits = lax.bitcast_convert_type(acc, jnp.int32)
            key = jnp.where(bits < 0, jnp.int32(-2147483648) - bits, bits)
            ukey = lax.bitcast_convert_type(key ^ jnp.int32(-2147483648),
                                            jnp.uint32)
            kbuf[pl.ds(i * 16, 16)] = ukey
            return carry

        lax.fori_loop(0, NCHUNK, score_chunk, jnp.int32(0))

        def _tot(v):
            parts = [v[l] for l in range(16)]
            while len(parts) > 1:
                parts = [parts[i] + parts[i + 1]
                         for i in range(0, len(parts), 2)]
            return parts[0]

        one = jnp.int32(1)
        zero = jnp.int32(0)

        def count_ge(tval):
            tv16 = jnp.full((16,), tval, jnp.uint32)

            def cbody(i, c):
                for u in range(4):
                    k = kbuf[pl.ds((i * 4 + u) * 16, 16)]
                    c = c + jnp.where(k >= tv16, one, zero)
                return c

            c = lax.fori_loop(0, NCHUNK // 4, cbody,
                              jnp.zeros((16,), jnp.int32))
            return _tot(c)

        def bis(it, carry):
            t, bit = carry
            cand = t | bit
            t = jnp.where(count_ge(cand) >= CUT, cand, t)
            return t, bit >> jnp.uint32(1)

        tsel, _ = lax.fori_loop(0, 32, bis,
                                (jnp.uint32(0), jnp.uint32(1 << 31)))

        tv16 = jnp.full((16,), tsel, jnp.uint32)

        def cgt_body(i, c):
            for u in range(4):
                k = kbuf[pl.ds((i * 4 + u) * 16, 16)]
                c = c + jnp.where(k > tv16, one, zero)
            return c

        count_gt = _tot(lax.fori_loop(0, NCHUNK // 4, cgt_body,
                                      jnp.zeros((16,), jnp.int32)))
        need_eq = jnp.int32(CUT) - count_gt

        lane = lax.iota(jnp.int32, 16)

        def mbody(i, rbefore):
            k = kbuf[pl.ds(i * 16, 16)]
            gt = k > tv16
            eq = k == tv16
            eqiv = jnp.where(eq, one, zero)
            eqtot = _tot(eqiv)

            def with_eq(_):
                r = need_eq - rbefore
                r_eff = jnp.minimum(jnp.maximum(r, zero), eqtot)
                r_m = jnp.where(r_eff <= 0, jnp.int32(999), r_eff)
                run = zero
                lc = jnp.int32(-1)
                for l in range(16):
                    run = run + eqiv[l]
                    lc = jnp.where((lc == -1) & (run == r_m),
                                   jnp.int32(l), lc)
                return lc

            lcut = lax.cond(eqtot > 0, with_eq,
                            lambda _: jnp.int32(-1), zero)
            sel_eq = eq & (lane <= jnp.full((16,), lcut, jnp.int32))
            selv = gt | sel_eq
            sbuf[pl.ds(i * 16, 16)] = jnp.where(selv, 1.0, 0.0)
            return rbefore + eqtot

        lax.fori_loop(0, NCHUNK, mbody, jnp.int32(0))
        pltpu.sync_copy(sbuf, sel_hbm.at[b])


def _select_topk_sc(xr):
    b = xr.shape[0]
    mesh = plsc.VectorSubcoreMesh(core_axis_name="c", subcore_axis_name="s")
    return pl.kernel(
        _sel_sc_body,
        out_type=jax.ShapeDtypeStruct((b, NP_), jnp.float32),
        mesh=mesh,
        scratch_types=[
            pltpu.VMEM((NHEAD, NP_), jnp.float32),
            pltpu.VMEM((NP_,), jnp.uint32),
            pltpu.VMEM((NP_,), jnp.float32),
            pltpu.VMEM((16,), jnp.int32),
        ],
    )(xr)


def _fix_kernel(sel_ref, img_ref, out_ref, mh_ref):
    j = pl.program_id(0)

    @pl.when(j == 0)
    def _expand_masks():
        p_g = lax.broadcasted_iota(jnp.int32, (FEAT, NP_), 1)
        i_g = lax.broadcasted_iota(jnp.int32, (FEAT, NP_), 0)
        G = ((p_g // FEAT) == i_g).astype(jnp.float32)
        p_h = lax.broadcasted_iota(jnp.int32, (NP_, FEAT), 0)
        j_h = lax.broadcasted_iota(jnp.int32, (NP_, FEAT), 1)
        H = ((p_h % FEAT) == j_h).astype(jnp.float32)
        k_t = lax.broadcasted_iota(jnp.int32, (FEAT, IMG), 0)
        c_t = lax.broadcasted_iota(jnp.int32, (FEAT, IMG), 1)
        ET = ((c_t // PATCH) == k_t).astype(jnp.float32)

        for b in range(sel_ref.shape[0]):
            srow = sel_ref[pl.ds(b, 1), :]
            m2 = jnp.dot(G * srow, H,
                         preferred_element_type=jnp.float32)
            mh_ref[b] = jnp.dot(m2, ET,
                                preferred_element_type=jnp.float32)

    rb = j // 3
    m8 = mh_ref[:, pl.ds(8 * rb, 8), :]
    mrows = jnp.repeat(m8, PATCH, axis=1)
    prod = img_ref[:, 0] * mrows
    out_ref[...] = jnp.reshape(prod, (prod.shape[0], COLW))


def kernel(x, input_images):
    B, C = input_images.shape[0], input_images.shape[1]
    xr = x[:, :, 0, 1:]
    selm = _select_topk_sc(xr)

    def img_idx(jg):
        return (0, jg % C, jg // C, 0)

    def out_idx(jg):
        return (0, (jg % C) * NRB + jg // C)

    out = pl.pallas_call(
        _fix_kernel,
        grid=(C * NRB,),
        in_specs=[
            pl.BlockSpec((B, NP_), lambda jg: (0, 0)),
            pl.BlockSpec((B, 1, RB, IMG), img_idx),
        ],
        out_specs=pl.BlockSpec((B, COLW), out_idx),
        out_shape=jax.ShapeDtypeStruct((B, C * IMG * IMG), jnp.float32),
        scratch_shapes=[pltpu.VMEM((B, FEAT, IMG), jnp.float32)],
    )(selm, input_images)
    return out


# speedup vs baseline: 1.0171x; 1.0171x over previous
import jax
import jax.numpy as jnp
from jax import lax
from jax.experimental import pallas as pl
from jax.experimental.pallas import tpu as pltpu
from jax.experimental.pallas import tpu_sc as plsc

PATCH = 16
FEAT = 24
NP_ = FEAT * FEAT
CUT = NP_ // 2
IMG = FEAT * PATCH
RB = 128
NRB = IMG // RB
COLW = RB * IMG
NCHUNK = NP_ // 16
NHEAD = 12


def _sel_sc_body(xr_hbm, sel_hbm, xbuf, kbuf, sbuf):
    wid = lax.axis_index("s") * 2 + lax.axis_index("c")

    @pl.when(wid < xr_hbm.shape[0])
    def _():
        b = wid
        pltpu.sync_copy(xr_hbm.at[b], xbuf)

        def score_chunk(i, carry):
            acc = xbuf[0, pl.ds(i * 16, 16)]
            for h in range(1, NHEAD):
                acc = acc + xbuf[h, pl.ds(i * 16, 16)]
            bits = lax.bitcast_convert_type(acc, jnp.int32)
            key = jnp.where(bits < 0, jnp.int32(-2147483648) - bits, bits)
            ukey = lax.bitcast_convert_type(key ^ jnp.int32(-2147483648),
                                            jnp.uint32)
            kbuf[pl.ds(i * 16, 16)] = ukey
            return carry

        lax.fori_loop(0, NCHUNK, score_chunk, jnp.int32(0))

        def _tot(v):
            parts = [v[l] for l in range(16)]
            while len(parts) > 1:
                parts = [parts[i] + parts[i + 1]
                         for i in range(0, len(parts), 2)]
            return parts[0]

        one = jnp.int32(1)
        zero = jnp.int32(0)

        def count_ge(tval):
            tv16 = jnp.full((16,), tval, jnp.uint32)

            def cbody(i, c):
                for u in range(4):
                    k = kbuf[pl.ds((i * 4 + u) * 16, 16)]
                    c = c + jnp.where(k >= tv16, one, zero)
                return c

            c = lax.fori_loop(0, NCHUNK // 4, cbody,
                              jnp.zeros((16,), jnp.int32))
            return _tot(c)

        def bis(it, carry):
            t, bit = carry
            cand = t | bit
            t = jnp.where(count_ge(cand) >= CUT, cand, t)
            return t, bit >> jnp.uint32(1)

        tsel, _ = lax.fori_loop(0, 32, bis,
                                (jnp.uint32(0), jnp.uint32(1 << 31)))

        tv16 = jnp.full((16,), tsel, jnp.uint32)

        def cgt_body(i, c):
            for u in range(4):
                k = kbuf[pl.ds((i * 4 + u) * 16, 16)]
                c = c + jnp.where(k > tv16, one, zero)
            return c

        count_gt = _tot(lax.fori_loop(0, NCHUNK // 4, cgt_body,
                                      jnp.zeros((16,), jnp.int32)))
        need_eq = jnp.int32(CUT) - count_gt

        lane = lax.iota(jnp.int32, 16)

        def mbody(i, rbefore):
            k = kbuf[pl.ds(i * 16, 16)]
            gt = k > tv16
            eq = k == tv16
            eqiv = jnp.where(eq, one, zero)
            eqtot = _tot(eqiv)

            def with_eq(_):
                r = need_eq - rbefore
                r_eff = jnp.minimum(jnp.maximum(r, zero), eqtot)
                r_m = jnp.where(r_eff <= 0, jnp.int32(999), r_eff)
                run = zero
                lc = jnp.int32(-1)
                for l in range(16):
                    run = run + eqiv[l]
                    lc = jnp.where((lc == -1) & (run == r_m),
                                   jnp.int32(l), lc)
                return lc

            lcut = lax.cond(eqtot > 0, with_eq,
                            lambda _: jnp.int32(-1), zero)
            sel_eq = eq & (lane <= jnp.full((16,), lcut, jnp.int32))
            selv = gt | sel_eq
            sbuf[pl.ds(i * 16, 16)] = jnp.where(selv, 1.0, 0.0)
            return rbefore + eqtot

        lax.fori_loop(0, NCHUNK, mbody, jnp.int32(0))
        pltpu.sync_copy(sbuf, sel_hbm.at[b])


def _select_topk_sc(xr):
    b = xr.shape[0]
    mesh = plsc.VectorSubcoreMesh(core_axis_name="c", subcore_axis_name="s")
    return pl.kernel(
        _sel_sc_body,
        out_type=jax.ShapeDtypeStruct((b, NP_), jnp.float32),
        mesh=mesh,
        scratch_types=[
            pltpu.VMEM((NHEAD, NP_), jnp.float32),
            pltpu.VMEM((NP_,), jnp.uint32),
            pltpu.VMEM((NP_,), jnp.float32),
        ],
    )(xr)


def _fix_kernel(sel_ref, img_ref, out_ref, mh_ref):
    j = pl.program_id(0)

    @pl.when(j == 0)
    def _expand_masks():
        p_g = lax.broadcasted_iota(jnp.int32, (FEAT, NP_), 1)
        i_g = lax.broadcasted_iota(jnp.int32, (FEAT, NP_), 0)
        G = ((p_g // FEAT) == i_g).astype(jnp.float32)
        p_h = lax.broadcasted_iota(jnp.int32, (NP_, FEAT), 0)
        j_h = lax.broadcasted_iota(jnp.int32, (NP_, FEAT), 1)
        H = ((p_h % FEAT) == j_h).astype(jnp.float32)
        k_t = lax.broadcasted_iota(jnp.int32, (FEAT, IMG), 0)
        c_t = lax.broadcasted_iota(jnp.int32, (FEAT, IMG), 1)
        ET = ((c_t // PATCH) == k_t).astype(jnp.float32)

        for b in range(sel_ref.shape[0]):
            srow = sel_ref[pl.ds(b, 1), :]
            m2 = jnp.dot(G * srow, H,
                         preferred_element_type=jnp.float32)
            mh_ref[b] = jnp.dot(m2, ET,
                                preferred_element_type=jnp.float32)

    @pl.when(j > 0)
    def _stream_block():
        jj = j - 1
        rb = jj // 3
        m8 = mh_ref[:, pl.ds(8 * rb, 8), :]
        mrows = jnp.repeat(m8, PATCH, axis=1)
        prod = img_ref[:, 0] * mrows
        out_ref[...] = jnp.reshape(prod, (prod.shape[0], COLW))


def kernel(x, input_images):
    B, C = input_images.shape[0], input_images.shape[1]
    xr = x[:, :, 0, 1:]
    selm = _select_topk_sc(xr)

    def img_idx(jg):
        jj = jnp.maximum(jg - 1, 0)
        return (0, jj % C, jj // C, 0)

    def out_idx(jg):
        jj = jnp.maximum(jg - 1, 0)
        return (0, (jj % C) * NRB + jj // C)

    out = pl.pallas_call(
        _fix_kernel,
        grid=(1 + C * NRB,),
        in_specs=[
            pl.BlockSpec((B, NP_), lambda jg: (0, 0)),
            pl.BlockSpec((B, 1, RB, IMG), img_idx),
        ],
        out_specs=pl.BlockSpec((B, COLW), out_idx),
        out_shape=jax.ShapeDtypeStruct((B, C * IMG * IMG), jnp.float32),
        scratch_shapes=[pltpu.VMEM((B, FEAT, IMG), jnp.float32)],
    )(selm, input_images)
    return out
